# Initial kernel scaffold; baseline (speedup 1.0000x reference)
import jax
import jax.numpy as jnp
from jax.experimental import pallas as pl


def kernel(movie_id, user_id, movie_genres, movie_table, user_table, genre_table, W1, b1, W2, b2):
    raise NotImplementedError("write your pallas kernel here")


# speedup vs baseline: 8.6985x; 8.6985x over previous
import functools

import jax
import jax.numpy as jnp
from jax import lax
from jax.experimental import pallas as pl
from jax.experimental.pallas import tpu as pltpu
from jax.experimental.pallas import tpu_sc as plsc

B = 4096
L = 20
LPAD = 32
EMB = 128
GEMB = 32
H1 = 256
H2 = 128

NC = 2
NS = 16
NW = NC * NS
BW = B // NW


def _sc_gather_pool(mi, ui, gflat, mtab, utab, gtab):
  mesh = plsc.VectorSubcoreMesh(core_axis_name="c", subcore_axis_name="s")

  @functools.partial(
      pl.kernel,
      mesh=mesh,
      compiler_params=pltpu.CompilerParams(use_tc_tiling_on_sc=False),
      out_type=[
          jax.ShapeDtypeStruct((B, EMB), jnp.float32),
          jax.ShapeDtypeStruct((B, EMB), jnp.float32),
          jax.ShapeDtypeStruct((B, GEMB), jnp.float32),
      ],
      scratch_types=[
          pltpu.VMEM((BW,), jnp.int32),
          pltpu.VMEM((BW,), jnp.int32),
          pltpu.VMEM((BW * L,), jnp.int32),
          pltpu.VMEM((BW, EMB), jnp.float32),
          pltpu.VMEM((BW, EMB), jnp.float32),
          pltpu.VMEM((BW * L, GEMB), jnp.float32),
          pltpu.VMEM((BW, GEMB), jnp.float32),
          pltpu.SemaphoreType.DMA,
          pltpu.SemaphoreType.DMA,
          pltpu.SemaphoreType.DMA,
      ],
  )
  def k(mi_hbm, ui_hbm, gflat_hbm, mtab_hbm, utab_hbm, gtab_hbm,
        mout_hbm, uout_hbm, gout_hbm,
        mi_v, ui_v, gi_v, mrows, urows, grows, gavg, sem_m, sem_u, sem_g):
    wid = lax.axis_index("s") * NC + lax.axis_index("c")
    base = wid * BW

    pltpu.sync_copy(mi_hbm.at[pl.ds(base, BW)], mi_v)
    pltpu.sync_copy(ui_hbm.at[pl.ds(base, BW)], ui_v)
    pltpu.sync_copy(gflat_hbm.at[pl.ds(base * L, BW * L)], gi_v)

    cp_m = pltpu.async_copy(mtab_hbm.at[mi_v], mrows, sem_m)
    cp_u = pltpu.async_copy(utab_hbm.at[ui_v], urows, sem_u)
    cp_g = pltpu.async_copy(gtab_hbm.at[gi_v], grows, sem_g)

    cp_g.wait()

    def row_body(i, carry):
      accs = []
      for half in range(GEMB // 16):
        acc = grows[i * L, pl.ds(half * 16, 16)]
        for l in range(1, L):
          acc = acc + grows[i * L + l, pl.ds(half * 16, 16)]
        accs.append(acc)
      for half in range(GEMB // 16):
        gavg[i, pl.ds(half * 16, 16)] = accs[half]
      return carry

    lax.fori_loop(0, BW, row_body, 0)

    cp_m.wait()
    cp_u.wait()
    pltpu.sync_copy(mrows, mout_hbm.at[pl.ds(base, BW)])
    pltpu.sync_copy(urows, uout_hbm.at[pl.ds(base, BW)])
    pltpu.sync_copy(gavg, gout_hbm.at[pl.ds(base, BW)])

  return k(mi, ui, gflat, mtab, utab, gtab)


def _mlp_body(m_ref, u_ref, g_ref, mg_ref, w1m_ref, w1u_ref, w1g_ref, b1_ref,
              w2_ref, b2_ref, out_ref):
  denom = jnp.sum((mg_ref[...] != 0).astype(jnp.float32), axis=1, keepdims=True)
  g = g_ref[...] / denom
  h = jnp.dot(m_ref[...], w1m_ref[...], preferred_element_type=jnp.float32)
  h = h + jnp.dot(u_ref[...], w1u_ref[...], preferred_element_type=jnp.float32)
  h = h + jnp.dot(g, w1g_ref[...], preferred_element_type=jnp.float32)
  h = jnp.maximum(h + b1_ref[...], 0.0)
  o = jnp.dot(h, w2_ref[...], preferred_element_type=jnp.float32)
  out_ref[...] = jnp.maximum(o + b2_ref[...], 0.0)


def _mlp(movie_e, user_e, genre_sum, mg_pad, W1m, W1u, W1g, b1, W2, b2):
  BB = 512
  grid = (B // BB,)
  return pl.pallas_call(
      _mlp_body,
      grid=grid,
      in_specs=[
          pl.BlockSpec((BB, EMB), lambda i: (i, 0)),
          pl.BlockSpec((BB, EMB), lambda i: (i, 0)),
          pl.BlockSpec((BB, GEMB), lambda i: (i, 0)),
          pl.BlockSpec((BB, LPAD), lambda i: (i, 0)),
          pl.BlockSpec((EMB, H1), lambda i: (0, 0)),
          pl.BlockSpec((EMB, H1), lambda i: (0, 0)),
          pl.BlockSpec((GEMB, H1), lambda i: (0, 0)),
          pl.BlockSpec((1, H1), lambda i: (0, 0)),
          pl.BlockSpec((H1, H2), lambda i: (0, 0)),
          pl.BlockSpec((1, H2), lambda i: (0, 0)),
      ],
      out_specs=pl.BlockSpec((BB, H2), lambda i: (i, 0)),
      out_shape=jax.ShapeDtypeStruct((B, H2), jnp.float32),
  )(movie_e, user_e, genre_sum, mg_pad, W1m, W1u, W1g, b1, W2, b2)


def kernel(movie_id, user_id, movie_genres, movie_table, user_table,
           genre_table, W1, b1, W2, b2):
  mi = movie_id.astype(jnp.int32)
  ui = user_id.astype(jnp.int32)
  mg = movie_genres.astype(jnp.int32)
  gflat = mg.reshape(-1)
  gpad = jnp.pad(mg, ((0, 0), (0, LPAD - L)))

  movie_e, user_e, genre_sum = _sc_gather_pool(
      mi, ui, gflat, movie_table, user_table, genre_table)

  W1m = W1[:EMB]
  W1u = W1[EMB:2 * EMB]
  W1g = W1[2 * EMB:]
  return _mlp(movie_e, user_e, genre_sum, gpad, W1m, W1u, W1g,
              b1.reshape(1, H1), W2, b2.reshape(1, H2))


# speedup vs baseline: 9.2735x; 1.0661x over previous
import functools

import jax
import jax.numpy as jnp
from jax import lax
from jax.experimental import pallas as pl
from jax.experimental.pallas import tpu as pltpu
from jax.experimental.pallas import tpu_sc as plsc

B = 4096
L = 20
EMB = 128
GEMB = 32
H1 = 256
H2 = 128

NC = 2
NS = 16
NW = NC * NS
BW = B // NW


def _sc_gather_pool(mi, ui, mg, mtab, utab, gtab):
  mesh = plsc.VectorSubcoreMesh(core_axis_name="c", subcore_axis_name="s")

  @functools.partial(
      pl.kernel,
      mesh=mesh,
      compiler_params=pltpu.CompilerParams(
          use_tc_tiling_on_sc=False, needs_layout_passes=False),
      out_type=[
          jax.ShapeDtypeStruct((B, EMB), jnp.float32),
          jax.ShapeDtypeStruct((B, EMB), jnp.float32),
          jax.ShapeDtypeStruct((B, GEMB), jnp.float32),
      ],
      scratch_types=[
          pltpu.VMEM((BW,), jnp.int32),
          pltpu.VMEM((BW,), jnp.int32),
          pltpu.VMEM((BW, L), jnp.int32),
          pltpu.VMEM((L, BW), jnp.int32),
          pltpu.VMEM((BW, EMB), jnp.float32),
          pltpu.VMEM((BW, EMB), jnp.float32),
          pltpu.VMEM((BW, GEMB), jnp.float32),
          pltpu.SemaphoreType.DMA,
          pltpu.SemaphoreType.DMA,
          pltpu.SemaphoreType.DMA,
      ],
  )
  def k(mi_hbm, ui_hbm, mg_hbm, mtab_hbm, utab_hbm, gtab_hbm,
        mout_hbm, uout_hbm, gout_hbm,
        mi_v, ui_v, gi_v, gi_t, mrows, urows, gacc, sem_m, sem_u, sem_g):
    wid = lax.axis_index("s") * NC + lax.axis_index("c")
    base = wid * BW

    pltpu.sync_copy(mi_hbm.at[pl.ds(base, BW)], mi_v)
    pltpu.sync_copy(ui_hbm.at[pl.ds(base, BW)], ui_v)
    pltpu.sync_copy(mg_hbm.at[pl.ds(base, BW)], gi_v)

    cp_m = pltpu.async_copy(mtab_hbm.at[mi_v], mrows, sem_m)
    cp_u = pltpu.async_copy(utab_hbm.at[ui_v], urows, sem_u)

    for l in range(L):
      col = jnp.full((16,), l, jnp.int32)
      for j in range(BW // 16):
        rows = lax.iota(jnp.int32, 16) + j * 16
        ids = plsc.load_gather(gi_v, [rows, col])
        gi_t[l, pl.ds(j * 16, 16)] = ids

    pltpu.async_copy(gtab_hbm.at[gi_t.at[0]], gacc, sem_g).wait()
    cps = [
        pltpu.async_copy(gtab_hbm.at[gi_t.at[l]], gacc, sem_g, add=True)
        for l in range(1, L)
    ]
    for cp in cps:
      cp.wait()

    cp_m.wait()
    cp_u.wait()
    pltpu.sync_copy(mrows, mout_hbm.at[pl.ds(base, BW)])
    pltpu.sync_copy(urows, uout_hbm.at[pl.ds(base, BW)])
    pltpu.sync_copy(gacc, gout_hbm.at[pl.ds(base, BW)])

  return k(mi, ui, mg, mtab, utab, gtab)


def _mlp_body(m_ref, u_ref, g_ref, mg_ref, w1_ref, b1_ref,
              w2_ref, b2_ref, out_ref):
  denom = jnp.sum((mg_ref[...] != 0).astype(jnp.float32), axis=1, keepdims=True)
  g = g_ref[...] / denom
  h = jnp.dot(m_ref[...], w1_ref[0:EMB, :], preferred_element_type=jnp.float32)
  h = h + jnp.dot(u_ref[...], w1_ref[EMB:2 * EMB, :],
                  preferred_element_type=jnp.float32)
  h = h + jnp.dot(g, w1_ref[2 * EMB:, :], preferred_element_type=jnp.float32)
  h = jnp.maximum(h + b1_ref[...], 0.0)
  o = jnp.dot(h, w2_ref[...], preferred_element_type=jnp.float32)
  out_ref[...] = jnp.maximum(o + b2_ref[...], 0.0)


def _mlp(movie_e, user_e, genre_sum, mg, W1, b1, W2, b2):
  BB = 512
  grid = (B // BB,)
  return pl.pallas_call(
      _mlp_body,
      grid=grid,
      in_specs=[
          pl.BlockSpec((BB, EMB), lambda i: (i, 0)),
          pl.BlockSpec((BB, EMB), lambda i: (i, 0)),
          pl.BlockSpec((BB, GEMB), lambda i: (i, 0)),
          pl.BlockSpec((BB, L), lambda i: (i, 0)),
          pl.BlockSpec((EMB + EMB + GEMB, H1), lambda i: (0, 0)),
          pl.BlockSpec((1, H1), lambda i: (0, 0)),
          pl.BlockSpec((H1, H2), lambda i: (0, 0)),
          pl.BlockSpec((1, H2), lambda i: (0, 0)),
      ],
      out_specs=pl.BlockSpec((BB, H2), lambda i: (i, 0)),
      out_shape=jax.ShapeDtypeStruct((B, H2), jnp.float32),
  )(movie_e, user_e, genre_sum, mg, W1, b1, W2, b2)


def kernel(movie_id, user_id, movie_genres, movie_table, user_table,
           genre_table, W1, b1, W2, b2):
  mi = movie_id.astype(jnp.int32)
  ui = user_id.astype(jnp.int32)
  mg = movie_genres.astype(jnp.int32)

  movie_e, user_e, genre_sum = _sc_gather_pool(
      mi, ui, mg, movie_table, user_table, genre_table)

  return _mlp(movie_e, user_e, genre_sum, mg, W1,
              b1.reshape(1, H1), W2, b2.reshape(1, H2))


# speedup vs baseline: 10.1479x; 1.0943x over previous
import functools

import jax
import jax.numpy as jnp
from jax import lax
from jax.experimental import pallas as pl
from jax.experimental.pallas import tpu as pltpu
from jax.experimental.pallas import tpu_sc as plsc

B = 4096
L = 20
EMB = 128
GEMB = 32
H1 = 256
H2 = 128

NC = 2
NS = 16
NW = NC * NS
BW = B // NW


def _sc_gather_pool(mi, ui, mg, mtab, utab, gtab):
  mesh = plsc.VectorSubcoreMesh(core_axis_name="c", subcore_axis_name="s")

  @functools.partial(
      pl.kernel,
      mesh=mesh,
      compiler_params=pltpu.CompilerParams(
          use_tc_tiling_on_sc=False, needs_layout_passes=False),
      out_type=[
          jax.ShapeDtypeStruct((B, EMB), jnp.float32),
          jax.ShapeDtypeStruct((B, EMB), jnp.float32),
          jax.ShapeDtypeStruct((B, GEMB), jnp.float32),
      ],
      scratch_types=[
          pltpu.VMEM((BW,), jnp.int32),
          pltpu.VMEM((BW,), jnp.int32),
          pltpu.VMEM((BW, L), jnp.int32),
          pltpu.VMEM((L, BW), jnp.int32),
          pltpu.VMEM((BW,), jnp.float32),
          pltpu.VMEM((BW, EMB), jnp.float32),
          pltpu.VMEM((BW, EMB), jnp.float32),
          pltpu.VMEM((BW, GEMB), jnp.float32),
          pltpu.SemaphoreType.DMA,
          pltpu.SemaphoreType.DMA,
          pltpu.SemaphoreType.DMA,
          pltpu.SemaphoreType.DMA,
      ],
  )
  def k(mi_hbm, ui_hbm, mg_hbm, mtab_hbm, utab_hbm, gtab_hbm,
        mout_hbm, uout_hbm, gout_hbm,
        mi_v, ui_v, gi_v, gi_t, rec_v, mrows, urows, gacc,
        sem_m, sem_u, sem_g, sem_i):
    wid = lax.axis_index("s") * NC + lax.axis_index("c")
    base = wid * BW

    cp_i1 = pltpu.async_copy(mi_hbm.at[pl.ds(base, BW)], mi_v, sem_i)
    cp_i2 = pltpu.async_copy(ui_hbm.at[pl.ds(base, BW)], ui_v, sem_i)
    cp_i3 = pltpu.async_copy(mg_hbm.at[pl.ds(base, BW)], gi_v, sem_i)

    zero = jnp.zeros((16,), jnp.float32)

    def zero_body(i, c):
      for half in range(GEMB // 16):
        gacc[i, pl.ds(half * 16, 16)] = zero
      return c

    lax.fori_loop(0, BW, zero_body, 0)

    cp_i1.wait()
    cp_m = pltpu.async_copy(mtab_hbm.at[mi_v], mrows, sem_m)
    cp_i2.wait()
    cp_u = pltpu.async_copy(utab_hbm.at[ui_v], urows, sem_u)
    cp_i3.wait()

    for l in range(L):
      col = jnp.full((16,), l, jnp.int32)
      for j in range(BW // 16):
        rows = lax.iota(jnp.int32, 16) + j * 16
        gi_t[l, pl.ds(j * 16, 16)] = plsc.load_gather(gi_v, [rows, col])

    cps = [
        pltpu.async_copy(gtab_hbm.at[gi_t.at[l]], gacc, sem_g, add=True)
        for l in range(L)
    ]

    one = jnp.ones((16,), jnp.float32)
    fzero = jnp.zeros((16,), jnp.float32)

    def cnt_body(j, c):
      cnt = jnp.zeros((16,), jnp.float32)
      for l in range(L):
        ids = gi_t[l, pl.ds(j * 16, 16)]
        cnt = cnt + jnp.where(ids != 0, one, fzero)
      rec_v[pl.ds(j * 16, 16)] = 1.0 / cnt
      return c

    lax.fori_loop(0, BW // 16, cnt_body, 0)

    cp_m.wait()
    pltpu.sync_copy(mrows, mout_hbm.at[pl.ds(base, BW)])
    cp_u.wait()
    pltpu.sync_copy(urows, uout_hbm.at[pl.ds(base, BW)])

    for cp in cps:
      cp.wait()

    def scale_body(i, c):
      r = plsc.load_gather(rec_v, [jnp.full((16,), i, jnp.int32)])
      for half in range(GEMB // 16):
        gacc[i, pl.ds(half * 16, 16)] = gacc[i, pl.ds(half * 16, 16)] * r
      return c

    lax.fori_loop(0, BW, scale_body, 0)

    pltpu.sync_copy(gacc, gout_hbm.at[pl.ds(base, BW)])

  return k(mi, ui, mg, mtab, utab, gtab)


def _mlp_body(m_ref, u_ref, g_ref, w1_ref, b1_ref, w2_ref, b2_ref, out_ref):
  h = jnp.dot(m_ref[...], w1_ref[0:EMB, :], preferred_element_type=jnp.float32)
  h = h + jnp.dot(u_ref[...], w1_ref[EMB:2 * EMB, :],
                  preferred_element_type=jnp.float32)
  h = h + jnp.dot(g_ref[...], w1_ref[2 * EMB:, :],
                  preferred_element_type=jnp.float32)
  h = jnp.maximum(h + b1_ref[...], 0.0)
  o = jnp.dot(h, w2_ref[...], preferred_element_type=jnp.float32)
  out_ref[...] = jnp.maximum(o + b2_ref[...], 0.0)


def _mlp(movie_e, user_e, genre_avg, W1, b1, W2, b2):
  BB = 1024
  grid = (B // BB,)
  return pl.pallas_call(
      _mlp_body,
      grid=grid,
      in_specs=[
          pl.BlockSpec((BB, EMB), lambda i: (i, 0)),
          pl.BlockSpec((BB, EMB), lambda i: (i, 0)),
          pl.BlockSpec((BB, GEMB), lambda i: (i, 0)),
          pl.BlockSpec((EMB + EMB + GEMB, H1), lambda i: (0, 0)),
          pl.BlockSpec((H1,), lambda i: (0,)),
          pl.BlockSpec((H1, H2), lambda i: (0, 0)),
          pl.BlockSpec((H2,), lambda i: (0,)),
      ],
      out_specs=pl.BlockSpec((BB, H2), lambda i: (i, 0)),
      out_shape=jax.ShapeDtypeStruct((B, H2), jnp.float32),
  )(movie_e, user_e, genre_avg, W1, b1, W2, b2)


def kernel(movie_id, user_id, movie_genres, movie_table, user_table,
           genre_table, W1, b1, W2, b2):
  mi = movie_id.astype(jnp.int32)
  ui = user_id.astype(jnp.int32)
  mg = movie_genres.astype(jnp.int32)

  movie_e, user_e, genre_avg = _sc_gather_pool(
      mi, ui, mg, movie_table, user_table, genre_table)

  return _mlp(movie_e, user_e, genre_avg, W1, b1, W2, b2)


# speedup vs baseline: 10.9619x; 1.0802x over previous
import functools

import jax
import jax.numpy as jnp
from jax import lax
from jax.experimental import pallas as pl
from jax.experimental.pallas import tpu as pltpu
from jax.experimental.pallas import tpu_sc as plsc

B = 4096
L = 20
EMB = 128
GEMB = 32
H1 = 256
H2 = 128

NC = 2
NS = 16
NW = NC * NS
BW = B // NW


def _sc_gather_pool(mi, ui, mg_flat, mtab, utab, gtab):
  mesh = plsc.VectorSubcoreMesh(core_axis_name="c", subcore_axis_name="s")

  @functools.partial(
      pl.kernel,
      mesh=mesh,
      compiler_params=pltpu.CompilerParams(
          use_tc_tiling_on_sc=False, needs_layout_passes=False),
      out_type=[
          jax.ShapeDtypeStruct((B, EMB), jnp.float32),
          jax.ShapeDtypeStruct((B, EMB), jnp.float32),
          jax.ShapeDtypeStruct((B, EMB), jnp.float32),
      ],
      scratch_types=[
          pltpu.VMEM((BW,), jnp.int32),
          pltpu.VMEM((BW,), jnp.int32),
          pltpu.VMEM((BW * L,), jnp.int32),
          pltpu.VMEM((L, BW), jnp.int32),
          pltpu.VMEM((BW,), jnp.float32),
          pltpu.VMEM((BW, EMB), jnp.float32),
          pltpu.VMEM((BW, EMB), jnp.float32),
          pltpu.VMEM((BW, GEMB), jnp.float32),
          pltpu.SemaphoreType.DMA,
          pltpu.SemaphoreType.DMA,
          pltpu.SemaphoreType.DMA,
          pltpu.SemaphoreType.DMA,
      ],
  )
  def k(mi_hbm, ui_hbm, mg_hbm, mtab_hbm, utab_hbm, gtab_hbm,
        mout_hbm, uout_hbm, gout_hbm,
        mi_v, ui_v, gi_v, gi_t, rec_v, mrows, urows, gacc,
        sem_m, sem_u, sem_g, sem_i):
    wid = lax.axis_index("s") * NC + lax.axis_index("c")
    base = wid * BW

    cp_i1 = pltpu.async_copy(mi_hbm.at[pl.ds(base, BW)], mi_v, sem_i)
    cp_i2 = pltpu.async_copy(ui_hbm.at[pl.ds(base, BW)], ui_v, sem_i)
    cp_i3 = pltpu.async_copy(mg_hbm.at[pl.ds(base * L, BW * L)], gi_v, sem_i)

    zero = jnp.zeros((16,), jnp.float32)

    def zero_body(i, c):
      for half in range(GEMB // 16):
        gacc[i, pl.ds(half * 16, 16)] = zero
      return c

    lax.fori_loop(0, BW, zero_body, 0)

    cp_i1.wait()
    cp_m = pltpu.async_copy(mtab_hbm.at[mi_v], mrows, sem_m)
    cp_i2.wait()
    cp_u = pltpu.async_copy(utab_hbm.at[ui_v], urows, sem_u)
    cp_i3.wait()

    def tr_body(l, c):
      for j in range(BW // 16):
        flat = (lax.iota(jnp.int32, 16) + j * 16) * L + l
        gi_t[l, pl.ds(j * 16, 16)] = plsc.load_gather(gi_v, [flat])
      return c

    lax.fori_loop(0, L, tr_body, 0)

    def add_body(l, c):
      pltpu.async_copy(gtab_hbm.at[gi_t.at[l]], gacc, sem_g, add=True)
      return c

    lax.fori_loop(0, L, add_body, 0)

    one = jnp.ones((16,), jnp.float32)
    fzero = jnp.zeros((16,), jnp.float32)

    def cnt_body(j, c):
      cnt = jnp.zeros((16,), jnp.float32)
      for l in range(L):
        ids = gi_t[l, pl.ds(j * 16, 16)]
        cnt = cnt + jnp.where(ids != 0, one, fzero)
      rec_v[pl.ds(j * 16, 16)] = 1.0 / cnt
      return c

    lax.fori_loop(0, BW // 16, cnt_body, 0)

    cp_m.wait()
    pltpu.sync_copy(mrows, mout_hbm.at[pl.ds(base, BW)])
    cp_u.wait()
    pltpu.sync_copy(urows, uout_hbm.at[pl.ds(base, BW)])

    def drain_body(l, c):
      pltpu.make_async_copy(gtab_hbm.at[gi_t.at[0]], gacc, sem_g).wait()
      return c

    lax.fori_loop(0, L, drain_body, 0)

    def scale_body(i, c):
      r = plsc.load_gather(rec_v, [jnp.full((16,), i, jnp.int32)])
      for half in range(GEMB // 16):
        gacc[i, pl.ds(half * 16, 16)] = gacc[i, pl.ds(half * 16, 16)] * r
      return c

    lax.fori_loop(0, BW, scale_body, 0)

    pltpu.sync_copy(gacc, gout_hbm.at[pl.ds(base, BW), pl.ds(0, GEMB)])

  return k(mi, ui, mg_flat, mtab, utab, gtab)


def _mlp_body(m_ref, u_ref, g_ref, w1_ref, b1_ref, w2_ref, b2_ref, out_ref):
  h = jnp.dot(m_ref[...], w1_ref[0:EMB, :], preferred_element_type=jnp.float32)
  h = h + jnp.dot(u_ref[...], w1_ref[EMB:2 * EMB, :],
                  preferred_element_type=jnp.float32)
  h = h + jnp.dot(g_ref[:, 0:GEMB], w1_ref[2 * EMB:, :],
                  preferred_element_type=jnp.float32)
  h = jnp.maximum(h + b1_ref[...], 0.0)
  o = jnp.dot(h, w2_ref[...], preferred_element_type=jnp.float32)
  out_ref[...] = jnp.maximum(o + b2_ref[...], 0.0)


def _mlp(movie_e, user_e, genre_avg, W1, b1, W2, b2):
  BB = 2048
  grid = (B // BB,)
  return pl.pallas_call(
      _mlp_body,
      grid=grid,
      in_specs=[
          pl.BlockSpec((BB, EMB), lambda i: (i, 0)),
          pl.BlockSpec((BB, EMB), lambda i: (i, 0)),
          pl.BlockSpec((BB, EMB), lambda i: (i, 0)),
          pl.BlockSpec((EMB + EMB + GEMB, H1), lambda i: (0, 0)),
          pl.BlockSpec((H1,), lambda i: (0,)),
          pl.BlockSpec((H1, H2), lambda i: (0, 0)),
          pl.BlockSpec((H2,), lambda i: (0,)),
      ],
      out_specs=pl.BlockSpec((BB, H2), lambda i: (i, 0)),
      out_shape=jax.ShapeDtypeStruct((B, H2), jnp.float32),
  )(movie_e, user_e, genre_avg, W1, b1, W2, b2)


def kernel(movie_id, user_id, movie_genres, movie_table, user_table,
           genre_table, W1, b1, W2, b2):
  mi = movie_id.astype(jnp.int32)
  ui = user_id.astype(jnp.int32)
  mg_flat = movie_genres.astype(jnp.int32).reshape(-1)

  movie_e, user_e, genre_avg = _sc_gather_pool(
      mi, ui, mg_flat, movie_table, user_table, genre_table)

  return _mlp(movie_e, user_e, genre_avg, W1, b1, W2, b2)


# speedup vs baseline: 11.5127x; 1.0502x over previous
import functools

import jax
import jax.numpy as jnp
from jax import lax
from jax.experimental import pallas as pl
from jax.experimental.pallas import tpu as pltpu
from jax.experimental.pallas import tpu_sc as plsc

B = 4096
L = 20
EMB = 128
GEMB = 32
GV = 1000
H1 = 256
H2 = 128

NC = 2
NS = 16
NW = NC * NS
BW = B // NW


def _sc_gather_pool(mi, ui, mg, mtab, utab, gtab):
  mesh = plsc.VectorSubcoreMesh(core_axis_name="c", subcore_axis_name="s")

  @functools.partial(
      pl.kernel,
      mesh=mesh,
      compiler_params=pltpu.CompilerParams(
          use_tc_tiling_on_sc=False, needs_layout_passes=False),
      out_type=[
          jax.ShapeDtypeStruct((B, EMB), jnp.float32),
          jax.ShapeDtypeStruct((B, EMB), jnp.float32),
          jax.ShapeDtypeStruct((B, EMB), jnp.float32),
      ],
      scratch_types=[
          pltpu.VMEM((BW,), jnp.int32),
          pltpu.VMEM((BW,), jnp.int32),
          pltpu.VMEM((BW, L), jnp.int32),
          pltpu.VMEM((GV, GEMB), jnp.float32),
          pltpu.VMEM((BW,), jnp.float32),
          pltpu.VMEM((BW, EMB), jnp.float32),
          pltpu.VMEM((BW, EMB), jnp.float32),
          pltpu.VMEM((BW, GEMB), jnp.float32),
          pltpu.SemaphoreType.DMA,
          pltpu.SemaphoreType.DMA,
          pltpu.SemaphoreType.DMA,
          pltpu.SemaphoreType.DMA,
      ],
  )
  def k(mi_hbm, ui_hbm, mg_hbm, mtab_hbm, utab_hbm, gtab_hbm,
        mout_hbm, uout_hbm, gout_hbm,
        mi_v, ui_v, gi_v, tab_v, rec_v, mrows, urows, gavg,
        sem_m, sem_u, sem_t, sem_i):
    wid = lax.axis_index("s") * NC + lax.axis_index("c")
    base = wid * BW

    cp_t = pltpu.async_copy(gtab_hbm, tab_v, sem_t)
    cp_i1 = pltpu.async_copy(mi_hbm.at[pl.ds(base, BW)], mi_v, sem_i)
    cp_i2 = pltpu.async_copy(ui_hbm.at[pl.ds(base, BW)], ui_v, sem_i)
    cp_i3 = pltpu.async_copy(mg_hbm.at[pl.ds(base, BW)], gi_v, sem_i)

    cp_i1.wait()
    cp_m = pltpu.async_copy(mtab_hbm.at[mi_v], mrows, sem_m)
    cp_i2.wait()
    cp_u = pltpu.async_copy(utab_hbm.at[ui_v], urows, sem_u)
    cp_i3.wait()

    one = jnp.ones((16,), jnp.float32)
    fzero = jnp.zeros((16,), jnp.float32)
    lanes = lax.iota(jnp.int32, 16)

    def cnt_body(j, c):
      rows = lanes + j * 16
      cnt = jnp.zeros((16,), jnp.float32)
      for l in range(L):
        ids = plsc.load_gather(gi_v, [rows, jnp.full((16,), l, jnp.int32)])
        cnt = cnt + jnp.where(ids != 0, one, fzero)
      rec_v[pl.ds(j * 16, 16)] = 1.0 / cnt
      return c

    lax.fori_loop(0, BW // 16, cnt_body, 0)

    cp_t.wait()

    hi = lanes + 16

    def pool_body(i, c):
      iv = jnp.full((16,), i, jnp.int32)
      acc0 = jnp.zeros((16,), jnp.float32)
      acc1 = jnp.zeros((16,), jnp.float32)
      for l in range(L):
        ids = plsc.load_gather(gi_v, [iv, jnp.full((16,), l, jnp.int32)])
        acc0 = acc0 + plsc.load_gather(tab_v, [ids, lanes])
        acc1 = acc1 + plsc.load_gather(tab_v, [ids, hi])
      r = plsc.load_gather(rec_v, [iv])
      gavg[i, pl.ds(0, 16)] = acc0 * r
      gavg[i, pl.ds(16, 16)] = acc1 * r
      return c

    lax.fori_loop(0, BW, pool_body, 0)

    cp_m.wait()
    pltpu.sync_copy(mrows, mout_hbm.at[pl.ds(base, BW)])
    cp_u.wait()
    pltpu.sync_copy(urows, uout_hbm.at[pl.ds(base, BW)])
    pltpu.sync_copy(gavg, gout_hbm.at[pl.ds(base, BW), pl.ds(0, GEMB)])

  return k(mi, ui, mg, mtab, utab, gtab)


def _mlp_body(m_ref, u_ref, g_ref, w1_ref, b1_ref, w2_ref, b2_ref, out_ref):
  h = jnp.dot(m_ref[...], w1_ref[0:EMB, :], preferred_element_type=jnp.float32)
  h = h + jnp.dot(u_ref[...], w1_ref[EMB:2 * EMB, :],
                  preferred_element_type=jnp.float32)
  h = h + jnp.dot(g_ref[:, 0:GEMB], w1_ref[2 * EMB:, :],
                  preferred_element_type=jnp.float32)
  h = jnp.maximum(h + b1_ref[...], 0.0)
  o = jnp.dot(h, w2_ref[...], preferred_element_type=jnp.float32)
  out_ref[...] = jnp.maximum(o + b2_ref[...], 0.0)


def _mlp(movie_e, user_e, genre_avg, W1, b1, W2, b2):
  BB = 2048
  grid = (B // BB,)
  return pl.pallas_call(
      _mlp_body,
      grid=grid,
      in_specs=[
          pl.BlockSpec((BB, EMB), lambda i: (i, 0)),
          pl.BlockSpec((BB, EMB), lambda i: (i, 0)),
          pl.BlockSpec((BB, EMB), lambda i: (i, 0)),
          pl.BlockSpec((EMB + EMB + GEMB, H1), lambda i: (0, 0)),
          pl.BlockSpec((H1,), lambda i: (0,)),
          pl.BlockSpec((H1, H2), lambda i: (0, 0)),
          pl.BlockSpec((H2,), lambda i: (0,)),
      ],
      out_specs=pl.BlockSpec((BB, H2), lambda i: (i, 0)),
      out_shape=jax.ShapeDtypeStruct((B, H2), jnp.float32),
  )(movie_e, user_e, genre_avg, W1, b1, W2, b2)


def kernel(movie_id, user_id, movie_genres, movie_table, user_table,
           genre_table, W1, b1, W2, b2):
  mi = movie_id.astype(jnp.int32)
  ui = user_id.astype(jnp.int32)
  mg = movie_genres.astype(jnp.int32)

  movie_e, user_e, genre_avg = _sc_gather_pool(
      mi, ui, mg, movie_table, user_table, genre_table)

  return _mlp(movie_e, user_e, genre_avg, W1, b1, W2, b2)
